# baseline (speedup 1.0000x reference)
import functools

import jax
import jax.numpy as jnp
from jax import lax
from jax.experimental import pallas as pl
from jax.experimental.pallas import tpu as pltpu
from jax.experimental.pallas import tpu_sc as plsc

D_MODEL = 1024
NC = 2
NS = 16
L = 16
NW = NC * NS
SCALE = 32.0
P = 8
NBUF = 3
G = 2
Q = 32


@functools.lru_cache(maxsize=None)
def _make_kernel(B: int, S: int, D: int):
    W = S // NW
    nwin = W // P
    rows = B * P
    nq = D // (Q * L)
    mesh = plsc.VectorSubcoreMesh(
        core_axis_name="c", subcore_axis_name="s", num_cores=NC, num_subcores=NS
    )

    @functools.partial(
        pl.kernel,
        out_type=jax.ShapeDtypeStruct((B * S, D), jnp.float32),
        mesh=mesh,
        scratch_types=[
            pltpu.VMEM((nwin, rows), jnp.int32),
            pltpu.VMEM((rows, D), jnp.float32),
            pltpu.VMEM((rows, D), jnp.float32),
            pltpu.VMEM((rows, D), jnp.float32),
            pltpu.VMEM((P, D), jnp.float32),
            pltpu.VMEM((P, D), jnp.float32),
            pltpu.VMEM((P, D), jnp.float32),
            pltpu.SemaphoreType.DMA,
            pltpu.SemaphoreType.DMA,
            pltpu.SemaphoreType.DMA,
            pltpu.SemaphoreType.DMA,
            pltpu.SemaphoreType.DMA,
            pltpu.SemaphoreType.DMA,
            pltpu.SemaphoreType.DMA,
            pltpu.SemaphoreType.DMA,
            pltpu.SemaphoreType.DMA,
        ],
    )
    def emb_kernel(idx_hbm, table_hbm, pos_hbm, out_hbm,
                   idx_v, g0, g1, g2, p0, p1, p2,
                   gs0, gs1, gs2, ps0, ps1, ps2, ss0, ss1, ss2):
        gb = (g0, g1, g2)
        pb = (p0, p1, p2)
        gsem = (gs0, gs1, gs2)
        psem = (ps0, ps1, ps2)
        ssem = (ss0, ss1, ss2)
        wid = lax.axis_index("s") * NC + lax.axis_index("c")
        pltpu.sync_copy(idx_hbm.at[wid], idx_v)

        def start_window(n):
            q = n % NBUF
            hg = pltpu.async_copy(table_hbm.at[idx_v.at[n]], gb[q], gsem[q])
            hp = pltpu.async_copy(
                pos_hbm.at[pl.ds(wid * W + n * P, P)], pb[q], psem[q])
            return hg, hp

        hg = [None] * nwin
        hp = [None] * nwin
        hs = [None] * nwin
        for n in range(G):
            hg[n], hp[n] = start_window(n)

        for j in range(nwin):
            q = j % NBUF
            hg[j].wait()
            hp[j].wait()

            def row(i, carry, _q=q):
                for h in range(nq):
                    base = h * (Q * L)
                    pv = [pb[_q][i, pl.ds(base + t * L, L)] for t in range(Q)]
                    for b in range(B):
                        r = b * P + i
                        for t in range(Q):
                            sl = pl.ds(base + t * L, L)
                            gb[_q][r, sl] = gb[_q][r, sl] * SCALE + pv[t]
                return carry

            lax.fori_loop(0, P, row, 0)

            hs[j] = [
                pltpu.async_copy(
                    gb[q].at[pl.ds(b * P, P)],
                    out_hbm.at[pl.ds(b * S + wid * W + j * P, P)],
                    ssem[q],
                )
                for b in range(B)
            ]

            n = j + G
            if n < nwin:
                if n >= NBUF:
                    for h in hs[n - NBUF]:
                        h.wait()
                hg[n], hp[n] = start_window(n)

        for c in range(nwin - NBUF, nwin):
            for h in hs[c]:
                h.wait()

    return emb_kernel


def kernel(input_token_vec, table, pos_encoding):
    B, S = input_token_vec.shape
    W = S // NW
    nwin = W // P
    idx = (input_token_vec.reshape(B, NW, nwin, P)
           .transpose(1, 2, 0, 3)
           .reshape(NW, nwin, B * P))
    out = _make_kernel(B, S, D_MODEL)(idx, table, pos_encoding)
    return out.reshape(B, S, D_MODEL)


# speedup vs baseline: 1.0152x; 1.0152x over previous
import functools

import jax
import jax.numpy as jnp
from jax import lax
from jax.experimental import pallas as pl
from jax.experimental.pallas import tpu as pltpu
from jax.experimental.pallas import tpu_sc as plsc

D_MODEL = 1024
NC = 2
NS = 16
L = 16
NW = NC * NS
SCALE = 32.0
P = 8
NBUF = 3
G = 2
Q = 16


@functools.lru_cache(maxsize=None)
def _make_kernel(B: int, S: int, D: int):
    W = S // NW
    nwin = W // P
    rows = B * P
    nq = D // (Q * L)
    mesh = plsc.VectorSubcoreMesh(
        core_axis_name="c", subcore_axis_name="s", num_cores=NC, num_subcores=NS
    )

    @functools.partial(
        pl.kernel,
        out_type=jax.ShapeDtypeStruct((B * S, D), jnp.float32),
        mesh=mesh,
        scratch_types=[
            pltpu.VMEM((nwin, rows), jnp.int32),
            pltpu.VMEM((rows, D), jnp.float32),
            pltpu.VMEM((rows, D), jnp.float32),
            pltpu.VMEM((rows, D), jnp.float32),
            pltpu.VMEM((P, D), jnp.float32),
            pltpu.VMEM((P, D), jnp.float32),
            pltpu.VMEM((P, D), jnp.float32),
            pltpu.SemaphoreType.DMA,
            pltpu.SemaphoreType.DMA,
            pltpu.SemaphoreType.DMA,
            pltpu.SemaphoreType.DMA,
            pltpu.SemaphoreType.DMA,
            pltpu.SemaphoreType.DMA,
            pltpu.SemaphoreType.DMA,
            pltpu.SemaphoreType.DMA,
            pltpu.SemaphoreType.DMA,
        ],
    )
    def emb_kernel(idx_hbm, table_hbm, pos_hbm, out_hbm,
                   idx_v, g0, g1, g2, p0, p1, p2,
                   gs0, gs1, gs2, ps0, ps1, ps2, ss0, ss1, ss2):
        gb = (g0, g1, g2)
        pb = (p0, p1, p2)
        gsem = (gs0, gs1, gs2)
        psem = (ps0, ps1, ps2)
        ssem = (ss0, ss1, ss2)
        wid = lax.axis_index("s") * NC + lax.axis_index("c")
        pltpu.sync_copy(idx_hbm.at[wid], idx_v)

        def start_window(n):
            q = n % NBUF
            hg = pltpu.async_copy(table_hbm.at[idx_v.at[n]], gb[q], gsem[q])
            hp = pltpu.async_copy(
                pos_hbm.at[pl.ds(wid * W + n * P, P)], pb[q], psem[q])
            return hg, hp

        hg = [None] * nwin
        hp = [None] * nwin
        hs = [None] * nwin
        for n in range(G):
            hg[n], hp[n] = start_window(n)

        for j in range(nwin):
            q = j % NBUF
            hg[j].wait()
            hp[j].wait()

            @plsc.parallel_loop(0, P)
            def row(i, _q=q):
                for h in range(nq):
                    base = h * (Q * L)
                    pv = [pb[_q][i, pl.ds(base + t * L, L)] for t in range(Q)]
                    for b in range(B):
                        r = b * P + i
                        for t in range(Q):
                            sl = pl.ds(base + t * L, L)
                            gb[_q][r, sl] = gb[_q][r, sl] * SCALE + pv[t]

            hs[j] = [
                pltpu.async_copy(
                    gb[q].at[pl.ds(b * P, P)],
                    out_hbm.at[pl.ds(b * S + wid * W + j * P, P)],
                    ssem[q],
                )
                for b in range(B)
            ]

            n = j + G
            if n < nwin:
                if n >= NBUF:
                    for h in hs[n - NBUF]:
                        h.wait()
                hg[n], hp[n] = start_window(n)

        for c in range(nwin - NBUF, nwin):
            for h in hs[c]:
                h.wait()

    return emb_kernel


def kernel(input_token_vec, table, pos_encoding):
    B, S = input_token_vec.shape
    W = S // NW
    nwin = W // P
    idx = (input_token_vec.reshape(B, NW, nwin, P)
           .transpose(1, 2, 0, 3)
           .reshape(NW, nwin, B * P))
    out = _make_kernel(B, S, D_MODEL)(idx, table, pos_encoding)
    return out.reshape(B, S, D_MODEL)
